# baseline (speedup 1.0000x reference)
import jax
import jax.numpy as jnp
from jax.experimental import pallas as pl
from jax.experimental.pallas import tpu as pltpu

_TEMP_INV = 10.0
_N = 2048
_R = 8
_BI = 256
_NK = _N // _BI


def _body(p_ref, t_ref, out_ref, acc_ref, tp_ref, tt_ref):
    r = pl.program_id(0)

    @pl.when(r == 0)
    def _():
        acc_ref[0] = 0.0

    tp_ref[0, :] = jnp.zeros((_N,), jnp.float32)
    tt_ref[0, :] = jnp.zeros((_N,), jnp.float32)

    ap = (p_ref[0, 0, :] * (0.5 * _TEMP_INV)).astype(jnp.bfloat16)
    at = (t_ref[0, 0, :] * (0.5 * _TEMP_INV)).astype(jnp.bfloat16)

    sxy = 0.0
    sxx = 0.0
    syy = 0.0
    for i in range(_NK):
        lo = i * _BI
        hi = (i + 1) * _BI
        w = _N - lo

        pi = ap[lo:hi].reshape(_BI, 1)
        ps = ap[lo:].reshape(1, w)
        ti = at[lo:hi].reshape(_BI, 1)
        ts = at[lo:].reshape(1, w)

        bp_h = jnp.tanh(pi - ps)
        bt_h = jnp.tanh(ti - ts)
        ones_w = jnp.ones((1, w), jnp.bfloat16)
        ones_row = jnp.ones((1, _BI), jnp.bfloat16)
        dims = (((1,), (0,)), ((), ()))
        rs_p = jax.lax.dot_general(ones_w, bp_h.T, dims,
                                   preferred_element_type=jnp.float32)
        rs_t = jax.lax.dot_general(ones_w, bt_h.T, dims,
                                   preferred_element_type=jnp.float32)
        cs_p = jax.lax.dot_general(ones_row, bp_h, dims,
                                   preferred_element_type=jnp.float32)
        cs_t = jax.lax.dot_general(ones_row, bt_h, dims,
                                   preferred_element_type=jnp.float32)

        xb = 0.5 * (tp_ref[0, lo:hi] + rs_p[0, :])
        yb = 0.5 * (tt_ref[0, lo:hi] + rs_t[0, :])
        if i < _NK - 1:
            tp_ref[0, hi:] -= cs_p[0, _BI:]
            tt_ref[0, hi:] -= cs_t[0, _BI:]

        sxy += jnp.sum(xb * yb)
        sxx += jnp.sum(xb * xb)
        syy += jnp.sum(yb * yb)

    vx = jnp.sqrt(sxx / _N + 1e-8)
    vy = jnp.sqrt(syy / _N + 1e-8)
    rho = (sxy / _N) / (vx * vy + 1e-8)
    acc_ref[0] += (1.0 - rho) / _R

    @pl.when(r == _R - 1)
    def _():
        out_ref[0, 0] = acc_ref[0]


def kernel(preds, targets):
    p3 = preds.reshape(_R, 1, _N)
    t3 = targets.reshape(_R, 1, _N)
    out = pl.pallas_call(
        _body,
        grid=(_R,),
        in_specs=[
            pl.BlockSpec((1, 1, _N), lambda r: (r, 0, 0)),
            pl.BlockSpec((1, 1, _N), lambda r: (r, 0, 0)),
        ],
        out_specs=pl.BlockSpec(memory_space=pltpu.SMEM),
        out_shape=jax.ShapeDtypeStruct((1, 1), jnp.float32),
        scratch_shapes=[
            pltpu.SMEM((1,), jnp.float32),
            pltpu.VMEM((1, _N), jnp.float32),
            pltpu.VMEM((1, _N), jnp.float32),
        ],
    )(p3, t3)
    return out[0, 0]
